# baseline (speedup 1.0000x reference)
import functools

import jax
import jax.numpy as jnp
from jax import lax
from jax.experimental import pallas as pl
from jax.experimental.pallas import tpu as pltpu
from jax.experimental.pallas import tpu_sc as plsc

NUM_ROWS = 1000000
DIM = 32
N = 16384
J = 26
B_TOTAL = N * J

NC = 2
NS = 16
NW = NC * NS
NPW = N // NW
BPW = NPW * J
TPW = NPW // 128
UNITS = J * TPW


def _make_kernel():
  mesh = plsc.VectorSubcoreMesh(core_axis_name="c", subcore_axis_name="s")

  @functools.partial(
      pl.kernel,
      mesh=mesh,
      compiler_params=pltpu.CompilerParams(
          use_tc_tiling_on_sc=False, needs_layout_passes=False),
      out_type=jax.ShapeDtypeStruct((J, DIM // 8, N // 128, 8, 128),
                                    jnp.float32),
      scratch_types=[
          pltpu.VMEM((BPW,), jnp.int32),
          pltpu.VMEM((128,), jnp.int32),
          pltpu.VMEM((128,), jnp.int32),
          pltpu.VMEM((128, DIM), jnp.float32),
          pltpu.VMEM((128, DIM), jnp.float32),
          pltpu.VMEM((DIM // 8, 8, 128), jnp.float32),
          pltpu.VMEM((DIM // 8, 8, 128), jnp.float32),
          pltpu.SemaphoreType.DMA,
          pltpu.SemaphoreType.DMA,
          pltpu.SemaphoreType.DMA,
          pltpu.SemaphoreType.DMA,
      ],
  )
  def gather_kernel(idx_hbm, table_hbm, out_hbm, idxv, ul0, ul1, gr0, gr1,
                    ot0, ot1, g0, g1, s0, s1):
    wid = lax.axis_index("s") * NC + lax.axis_index("c")
    ulist = (ul0, ul1)
    grows = (gr0, gr1)
    otile = (ot0, ot1)
    gsem = (g0, g1)
    ssem = (s0, s1)

    lane = jax.lax.iota(jnp.int32, 16)
    lane_j = lane * J
    row_base = [jnp.full((16,), v * 16, jnp.int32) + lane for v in range(8)]
    col_const = [jnp.full((16,), d, jnp.int32) for d in range(DIM)]

    pltpu.sync_copy(idx_hbm.at[pl.ds(wid * BPW, BPW)], idxv)

    def build_ulist(b, u):
      j = u // TPW
      tl = u - j * TPW
      base = tl * (128 * J) + j
      for v in range(8):
        pos = lane_j + (base + v * 16 * J)
        vals = plsc.load_gather(idxv, [pos])
        ulist[b][pl.ds(v * 16, 16)] = vals

    def start_gather(b):
      return pltpu.async_copy(table_hbm.at[ulist[b]], grows[b], gsem[b])

    def wait_gather(b):
      pltpu.make_async_copy(table_hbm.at[ulist[b]], grows[b], gsem[b]).wait()

    def transpose(b):
      for d in range(DIM):
        for v in range(8):
          vals = plsc.load_gather(grows[b], [row_base[v], col_const[d]])
          otile[b][d // 8, d % 8, pl.ds(v * 16, 16)] = vals

    def start_store(b, u):
      j = u // TPW
      tl = u - j * TPW
      tc = wid * TPW + tl
      return pltpu.async_copy(otile[b], out_hbm.at[j, :, tc], ssem[b])

    def wait_store(b, u):
      j = u // TPW
      tl = u - j * TPW
      tc = wid * TPW + tl
      pltpu.make_async_copy(otile[b], out_hbm.at[j, :, tc], ssem[b]).wait()

    build_ulist(0, 0)
    start_gather(0)
    build_ulist(1, 1)
    start_gather(1)
    for u in (0, 1):
      b = u & 1
      wait_gather(b)
      transpose(b)
      start_store(b, u)
      build_ulist(b, u + 2)
      start_gather(b)

    def body(o, carry):
      for b in range(2):
        u = 2 * o + b
        wait_gather(b)
        wait_store(b, u)
        transpose(b)
        start_store(b, u)
        build_ulist(b, u + 2)
        start_gather(b)
      return carry

    lax.fori_loop(1, (UNITS - 2) // 2, body, 0)

    for u in (UNITS - 2, UNITS - 1):
      b = u & 1
      wait_gather(b)
      wait_store(b, u)
      transpose(b)
      start_store(b, u)
    for u in (UNITS - 2, UNITS - 1):
      wait_store(u & 1, u)

  return gather_kernel


_gather = _make_kernel()


@jax.jit
def kernel(idx, embeddings):
  idx_flat = idx.reshape(B_TOTAL)
  out5 = _gather(idx_flat, embeddings)
  return out5.transpose(2, 4, 0, 1, 3).reshape(N, J, DIM)


# speedup vs baseline: 1.3731x; 1.3731x over previous
import functools

import jax
import jax.numpy as jnp
from jax import lax
from jax.experimental import pallas as pl
from jax.experimental.pallas import tpu as pltpu
from jax.experimental.pallas import tpu_sc as plsc

NUM_ROWS = 1000000
DIM = 32
N = 16384
J = 26
B_TOTAL = N * J

NC = 2
NS = 16
NW = NC * NS
NPW = N // NW
BPW = NPW * J
TPW = NPW // 128
UNITS = J


def _make_kernel():
  mesh = plsc.VectorSubcoreMesh(core_axis_name="c", subcore_axis_name="s")

  @functools.partial(
      pl.kernel,
      mesh=mesh,
      compiler_params=pltpu.CompilerParams(
          use_tc_tiling_on_sc=False, needs_layout_passes=False),
      out_type=jax.ShapeDtypeStruct((J, DIM // 8, N // 128, 8, 128),
                                    jnp.float32),
      scratch_types=[
          pltpu.VMEM((BPW,), jnp.int32),
          pltpu.VMEM((NPW,), jnp.int32),
          pltpu.VMEM((NPW,), jnp.int32),
          pltpu.VMEM((NPW, DIM), jnp.float32),
          pltpu.VMEM((NPW, DIM), jnp.float32),
          pltpu.VMEM((DIM // 8, TPW, 8, 128), jnp.float32),
          pltpu.VMEM((DIM // 8, TPW, 8, 128), jnp.float32),
          pltpu.SemaphoreType.DMA,
          pltpu.SemaphoreType.DMA,
          pltpu.SemaphoreType.DMA,
          pltpu.SemaphoreType.DMA,
      ],
  )
  def gather_kernel(idx_hbm, table_hbm, out_hbm, idxv, ul0, ul1, gr0, gr1,
                    ot0, ot1, g0, g1, s0, s1):
    wid = lax.axis_index("s") * NC + lax.axis_index("c")
    ulist = (ul0, ul1)
    grows = (gr0, gr1)
    otile = (ot0, ot1)
    gsem = (g0, g1)
    ssem = (s0, s1)

    lane = jax.lax.iota(jnp.int32, 16)
    lane_j = lane * J

    pltpu.sync_copy(idx_hbm.at[pl.ds(wid * BPW, BPW)], idxv)

    def build_ulist(b, j):
      for s in range(NPW // 16):
        pos = lane_j + (s * 16 * J + j)
        vals = plsc.load_gather(idxv, [pos])
        ulist[b][pl.ds(s * 16, 16)] = vals

    def start_gather(b):
      return pltpu.async_copy(table_hbm.at[ulist[b]], grows[b], gsem[b])

    def wait_gather(b):
      pltpu.make_async_copy(table_hbm.at[ulist[b]], grows[b], gsem[b]).wait()

    def transpose(b):
      otf = otile[b]
      zerov = lane & 0

      def istep(i, carry):
        diag = (lane + i) & 15
        for half in range(2):
          m = diag + (half * 16)
          a = m >> 3
          r = m & 7
          for s in range(NPW // 16):
            rowv = lane + (s * 16)
            cv = lane + ((s % 8) * 16)
            tlv = zerov + (s // 8)
            vals = plsc.load_gather(grows[b], [rowv, m])
            plsc.store_scatter(otf, [a, tlv, r, cv], vals)
        return carry

      lax.fori_loop(0, 16, istep, 0)

    def start_store(b, j):
      return pltpu.async_copy(
          otile[b], out_hbm.at[j, :, pl.ds(wid * TPW, TPW)], ssem[b])

    def wait_store(b, j):
      pltpu.make_async_copy(
          otile[b], out_hbm.at[j, :, pl.ds(wid * TPW, TPW)], ssem[b]).wait()

    build_ulist(0, 0)
    start_gather(0)
    build_ulist(1, 1)
    start_gather(1)
    for j in (0, 1):
      b = j & 1
      wait_gather(b)
      transpose(b)
      start_store(b, j)
      build_ulist(b, j + 2)
      start_gather(b)

    def body(o, carry):
      for b in range(2):
        j = 2 * o + b
        wait_gather(b)
        wait_store(b, j)
        transpose(b)
        start_store(b, j)
        build_ulist(b, j + 2)
        start_gather(b)
      return carry

    lax.fori_loop(1, (UNITS - 2) // 2, body, 0)

    for j in (UNITS - 2, UNITS - 1):
      b = j & 1
      wait_gather(b)
      wait_store(b, j)
      transpose(b)
      start_store(b, j)
    for j in (UNITS - 2, UNITS - 1):
      wait_store(j & 1, j)

  return gather_kernel


_gather = _make_kernel()


@jax.jit
def kernel(idx, embeddings):
  idx_flat = idx.reshape(B_TOTAL)
  out5 = _gather(idx_flat, embeddings)
  return out5.transpose(2, 4, 0, 1, 3).reshape(N, J, DIM)


# speedup vs baseline: 1.3824x; 1.0068x over previous
import functools

import jax
import jax.numpy as jnp
from jax import lax
from jax.experimental import pallas as pl
from jax.experimental.pallas import tpu as pltpu
from jax.experimental.pallas import tpu_sc as plsc

NUM_ROWS = 1000000
DIM = 32
N = 16384
J = 26
B_TOTAL = N * J

NC = 2
NS = 16
NW = NC * NS
NPW = N // NW
BPW = NPW * J
TPW = NPW // 128
UNITS = J * TPW


def _make_kernel():
  mesh = plsc.VectorSubcoreMesh(core_axis_name="c", subcore_axis_name="s")

  @functools.partial(
      pl.kernel,
      mesh=mesh,
      compiler_params=pltpu.CompilerParams(
          use_tc_tiling_on_sc=False, needs_layout_passes=False),
      out_type=jax.ShapeDtypeStruct((J, DIM // 8, N // 128, 8, 128),
                                    jnp.float32),
      scratch_types=[
          pltpu.VMEM((BPW,), jnp.int32),
          pltpu.VMEM((128,), jnp.int32),
          pltpu.VMEM((128,), jnp.int32),
          pltpu.VMEM((128, DIM), jnp.float32),
          pltpu.VMEM((128, DIM), jnp.float32),
          pltpu.VMEM((DIM // 8, 8, 128), jnp.float32),
          pltpu.VMEM((DIM // 8, 8, 128), jnp.float32),
          pltpu.SemaphoreType.DMA,
          pltpu.SemaphoreType.DMA,
          pltpu.SemaphoreType.DMA,
          pltpu.SemaphoreType.DMA,
      ],
  )
  def gather_kernel(idx_hbm, table_hbm, out_hbm, idxv, ul0, ul1, gr0, gr1,
                    ot0, ot1, g0, g1, s0, s1):
    wid = lax.axis_index("s") * NC + lax.axis_index("c")
    ulist = (ul0, ul1)
    grows = (gr0, gr1)
    otile = (ot0, ot1)
    gsem = (g0, g1)
    ssem = (s0, s1)

    lane = jax.lax.iota(jnp.int32, 16)
    lane_j = lane * J

    pltpu.sync_copy(idx_hbm.at[pl.ds(wid * BPW, BPW)], idxv)

    def build_ulist(b, u):
      j = u // TPW
      tl = u - j * TPW
      base = tl * (128 * J) + j
      for v in range(8):
        pos = lane_j + (base + v * 16 * J)
        vals = plsc.load_gather(idxv, [pos])
        ulist[b][pl.ds(v * 16, 16)] = vals

    def start_gather(b):
      return pltpu.async_copy(table_hbm.at[ulist[b]], grows[b], gsem[b])

    def wait_gather(b):
      pltpu.make_async_copy(table_hbm.at[ulist[b]], grows[b], gsem[b]).wait()

    def transpose(b):
      otf = otile[b]

      def istep(i, carry):
        diag = (lane + i) & 15
        for half in range(2):
          m = diag + (half * 16)
          a = m >> 3
          r = m & 7
          for v in range(8):
            rowv = lane + (v * 16)
            vals = plsc.load_gather(grows[b], [rowv, m])
            plsc.store_scatter(otf, [a, r, rowv], vals)
        return carry

      lax.fori_loop(0, 16, istep, 0)

    def start_store(b, u):
      j = u // TPW
      tl = u - j * TPW
      tc = wid * TPW + tl
      return pltpu.async_copy(otile[b], out_hbm.at[j, :, tc], ssem[b])

    def wait_store(b, u):
      j = u // TPW
      tl = u - j * TPW
      tc = wid * TPW + tl
      pltpu.make_async_copy(otile[b], out_hbm.at[j, :, tc], ssem[b]).wait()

    build_ulist(0, 0)
    start_gather(0)
    build_ulist(1, 1)
    start_gather(1)
    for u in (0, 1):
      b = u & 1
      wait_gather(b)
      transpose(b)
      start_store(b, u)
      build_ulist(b, u + 2)
      start_gather(b)

    def body(o, carry):
      for b in range(2):
        u = 2 * o + b
        wait_gather(b)
        wait_store(b, u)
        transpose(b)
        start_store(b, u)
        build_ulist(b, u + 2)
        start_gather(b)
      return carry

    lax.fori_loop(1, (UNITS - 2) // 2, body, 0)

    for u in (UNITS - 2, UNITS - 1):
      b = u & 1
      wait_gather(b)
      wait_store(b, u)
      transpose(b)
      start_store(b, u)
    for u in (UNITS - 2, UNITS - 1):
      wait_store(u & 1, u)

  return gather_kernel


_gather = _make_kernel()


@jax.jit
def kernel(idx, embeddings):
  idx_flat = idx.reshape(B_TOTAL)
  out5 = _gather(idx_flat, embeddings)
  return out5.transpose(2, 4, 0, 1, 3).reshape(N, J, DIM)
